# baseline (device time: 49301 ns/iter reference)
import jax
import jax.numpy as jnp
from jax import lax
from jax.experimental import pallas as pl
from jax.experimental.pallas import tpu as pltpu

N_DEV = 4


def kernel(x, router_W, route_idx, expert_W, shared_W):
    n_tok, d_model = x.shape
    e_loc, _, d_ff = expert_W.shape
    e_total = e_loc * N_DEV

    def body(x_ref, rw_ref, idx_ref, ew_ref, sw_ref, out_ref,
             buf, send_sems, recv_sems):
        my = lax.axis_index("i")
        left = lax.rem(my + (N_DEV - 1), N_DEV)
        right = lax.rem(my + 1, N_DEV)

        barrier = pltpu.get_barrier_semaphore()
        pl.semaphore_signal(barrier, inc=1, device_id=(left,),
                            device_id_type=pl.DeviceIdType.MESH)
        pl.semaphore_signal(barrier, inc=1, device_id=(right,),
                            device_id_type=pl.DeviceIdType.MESH)
        pl.semaphore_wait(barrier, 2)

        buf[0, :, :, :] = ew_ref[:, :, :].astype(jnp.bfloat16)

        xf = x_ref[:, :]
        scores = jnp.dot(xf, rw_ref[:, :], preferred_element_type=jnp.float32)
        smax = jnp.max(scores, axis=-1, keepdims=True)
        e_sc = jnp.exp(scores - smax)
        probs = e_sc / jnp.sum(e_sc, axis=-1, keepdims=True)
        eids = lax.broadcasted_iota(jnp.int32, (n_tok, e_total), 1)
        chosen = jnp.sum(jnp.where(eids == idx_ref[:, :], probs, 0.0),
                         axis=-1, keepdims=True)

        xs = (xf * chosen).astype(jnp.bfloat16)
        xb = xf.astype(jnp.bfloat16)

        acc = jnp.dot(xb, sw_ref[:, :].astype(jnp.bfloat16),
                      preferred_element_type=jnp.float32)

        def contrib(acc, slot, origin):
            for j in range(e_loc):
                e_g = origin * e_loc + j
                xm = jnp.where(idx_ref[:, :] == e_g, xs,
                               jnp.zeros((), jnp.bfloat16))
                acc = acc + jnp.dot(xm, buf[slot, j],
                                    preferred_element_type=jnp.float32)
            return acc

        for h in range(N_DEV - 1):
            rdma = pltpu.make_async_remote_copy(
                src_ref=buf.at[h],
                dst_ref=buf.at[h + 1],
                send_sem=send_sems.at[h],
                recv_sem=recv_sems.at[h],
                device_id=(right,),
                device_id_type=pl.DeviceIdType.MESH,
            )
            rdma.start()
            acc = contrib(acc, h, lax.rem(my + (N_DEV - h), N_DEV))
            rdma.wait()
        acc = contrib(acc, N_DEV - 1, lax.rem(my + 1, N_DEV))

        out_ref[:, :] = acc

    return pl.pallas_call(
        body,
        out_shape=jax.ShapeDtypeStruct((n_tok, d_ff), jnp.float32),
        in_specs=[pl.BlockSpec(memory_space=pltpu.VMEM)] * 5,
        out_specs=pl.BlockSpec(memory_space=pltpu.VMEM),
        scratch_shapes=[
            pltpu.VMEM((N_DEV, e_loc, d_model, d_ff), jnp.bfloat16),
            pltpu.SemaphoreType.DMA((N_DEV - 1,)),
            pltpu.SemaphoreType.DMA((N_DEV - 1,)),
        ],
        compiler_params=pltpu.CompilerParams(collective_id=0),
    )(x, router_W, route_idx, expert_W, shared_W)


# device time: 32270 ns/iter; 1.5278x vs baseline; 1.5278x over previous
import jax
import jax.numpy as jnp
from jax import lax
from jax.experimental import pallas as pl
from jax.experimental.pallas import tpu as pltpu

N_DEV = 4
E_HALF = 2


def kernel(x, router_W, route_idx, expert_W, shared_W):
    n_tok, d_model = x.shape
    e_loc, _, d_ff = expert_W.shape
    e_total = e_loc * N_DEV

    def body(x_ref, rw_ref, idx_ref, ew_ref, sw_ref, out_ref,
             buf, send_sems, recv_sems):
        my = lax.axis_index("i")
        left = lax.rem(my + (N_DEV - 1), N_DEV)
        right = lax.rem(my + 1, N_DEV)

        barrier = pltpu.get_barrier_semaphore()
        pl.semaphore_signal(barrier, inc=1, device_id=(left,),
                            device_id_type=pl.DeviceIdType.MESH)
        pl.semaphore_signal(barrier, inc=1, device_id=(right,),
                            device_id_type=pl.DeviceIdType.MESH)

        buf[0, 0, :, :, :] = ew_ref[0:E_HALF, :, :].astype(jnp.bfloat16)
        buf[1, 0, :, :, :] = ew_ref[E_HALF:e_loc, :, :].astype(jnp.bfloat16)

        xf = x_ref[:, :]
        scores = jnp.dot(xf, rw_ref[:, :], preferred_element_type=jnp.float32)
        smax = jnp.max(scores, axis=-1, keepdims=True)
        e_sc = jnp.exp(scores - smax)
        probs = e_sc / jnp.sum(e_sc, axis=-1, keepdims=True)
        eids = lax.broadcasted_iota(jnp.int32, (n_tok, e_total), 1)
        chosen = jnp.sum(jnp.where(eids == idx_ref[:, :], probs, 0.0),
                         axis=-1, keepdims=True)

        xs = (xf * chosen).astype(jnp.bfloat16)
        xb = xf.astype(jnp.bfloat16)

        pl.semaphore_wait(barrier, 2)

        def contrib(acc, d, slot, origin):
            for j in range(E_HALF):
                e_g = origin * e_loc + d * E_HALF + j
                xm = jnp.where(idx_ref[:, :] == e_g, xs,
                               jnp.zeros((), jnp.bfloat16))
                acc = acc + jnp.dot(xm, buf[d, slot, j],
                                    preferred_element_type=jnp.float32)
            return acc

        acc = jnp.zeros((n_tok, d_ff), jnp.float32)
        for h in range(N_DEV - 1):
            rdma_cw = pltpu.make_async_remote_copy(
                src_ref=buf.at[0, h],
                dst_ref=buf.at[0, h + 1],
                send_sem=send_sems.at[0, h],
                recv_sem=recv_sems.at[0, h],
                device_id=(right,),
                device_id_type=pl.DeviceIdType.MESH,
            )
            rdma_ccw = pltpu.make_async_remote_copy(
                src_ref=buf.at[1, h],
                dst_ref=buf.at[1, h + 1],
                send_sem=send_sems.at[1, h],
                recv_sem=recv_sems.at[1, h],
                device_id=(left,),
                device_id_type=pl.DeviceIdType.MESH,
            )
            rdma_cw.start()
            rdma_ccw.start()
            if h == 0:
                acc = acc + jnp.dot(xb, sw_ref[:, :].astype(jnp.bfloat16),
                                    preferred_element_type=jnp.float32)
            acc = contrib(acc, 0, h, lax.rem(my + (N_DEV - h), N_DEV))
            acc = contrib(acc, 1, h, lax.rem(my + h, N_DEV))
            rdma_cw.wait()
            rdma_ccw.wait()
        acc = contrib(acc, 0, N_DEV - 1, lax.rem(my + 1, N_DEV))
        acc = contrib(acc, 1, N_DEV - 1, lax.rem(my + (N_DEV - 1), N_DEV))

        out_ref[:, :] = acc

    return pl.pallas_call(
        body,
        out_shape=jax.ShapeDtypeStruct((n_tok, d_ff), jnp.float32),
        in_specs=[pl.BlockSpec(memory_space=pltpu.VMEM)] * 5,
        out_specs=pl.BlockSpec(memory_space=pltpu.VMEM),
        scratch_shapes=[
            pltpu.VMEM((2, N_DEV, E_HALF, d_model, d_ff), jnp.bfloat16),
            pltpu.SemaphoreType.DMA((2, N_DEV - 1)),
            pltpu.SemaphoreType.DMA((2, N_DEV - 1)),
        ],
        compiler_params=pltpu.CompilerParams(collective_id=0),
    )(x, router_W, route_idx, expert_W, shared_W)


# device time: 30470 ns/iter; 1.6180x vs baseline; 1.0591x over previous
import jax
import jax.numpy as jnp
from jax import lax
from jax.experimental import pallas as pl
from jax.experimental.pallas import tpu as pltpu

N_DEV = 4
E_HALF = 2


def kernel(x, router_W, route_idx, expert_W, shared_W):
    n_tok, d_model = x.shape
    e_loc, _, d_ff = expert_W.shape
    e_total = e_loc * N_DEV

    def body(x_ref, rw_ref, idx_ref, ew_ref, sw_ref, out_ref,
             buf, send_sems, recv_sems):
        my = lax.axis_index("i")
        left = lax.rem(my + (N_DEV - 1), N_DEV)
        right = lax.rem(my + 1, N_DEV)

        barrier = pltpu.get_barrier_semaphore()
        pl.semaphore_signal(barrier, inc=1, device_id=(left,),
                            device_id_type=pl.DeviceIdType.MESH)
        pl.semaphore_signal(barrier, inc=1, device_id=(right,),
                            device_id_type=pl.DeviceIdType.MESH)

        buf[0, 0, :, :, :] = ew_ref[0:E_HALF, :, :].astype(jnp.bfloat16)
        buf[1, 0, :, :, :] = ew_ref[E_HALF:e_loc, :, :].astype(jnp.bfloat16)

        xf = x_ref[:, :]
        scores = jnp.dot(xf, rw_ref[:, :], preferred_element_type=jnp.float32)
        smax = jnp.max(scores, axis=-1, keepdims=True)
        e_sc = jnp.exp(scores - smax)
        probs = e_sc / jnp.sum(e_sc, axis=-1, keepdims=True)
        eids = lax.broadcasted_iota(jnp.int32, (n_tok, e_total), 1)
        chosen = jnp.sum(jnp.where(eids == idx_ref[:, :], probs, 0.0),
                         axis=-1, keepdims=True)

        xs = (xf * chosen).astype(jnp.bfloat16)
        xb = xf.astype(jnp.bfloat16)

        pl.semaphore_wait(barrier, 2)

        def contrib(acc, d, slot, origin):
            for j in range(E_HALF):
                e_g = origin * e_loc + d * E_HALF + j
                xm = jnp.where(idx_ref[:, :] == e_g, xs,
                               jnp.zeros((), jnp.bfloat16))
                acc = acc + jnp.dot(xm, buf[d, slot, j],
                                    preferred_element_type=jnp.float32)
            return acc

        def piece_rdma(d, h, j):
            return pltpu.make_async_remote_copy(
                src_ref=buf.at[d, h, j],
                dst_ref=buf.at[d, h + 1, j],
                send_sem=send_sems.at[d, h, j],
                recv_sem=recv_sems.at[d, h, j],
                device_id=(right,) if d == 0 else (left,),
                device_id_type=pl.DeviceIdType.MESH,
            )

        rdmas = {}
        for d in (0, 1):
            for j in range(E_HALF):
                rdmas[(d, 0, j)] = piece_rdma(d, 0, j)
                rdmas[(d, 0, j)].start()

        acc = jnp.dot(xb, sw_ref[:, :].astype(jnp.bfloat16),
                      preferred_element_type=jnp.float32)
        acc = contrib(acc, 0, 0, my)
        acc = contrib(acc, 1, 0, my)

        for h in range(1, N_DEV - 1):
            for d in (0, 1):
                for j in range(E_HALF):
                    rdmas[(d, h - 1, j)].wait_recv()
                    rdmas[(d, h, j)] = piece_rdma(d, h, j)
                    rdmas[(d, h, j)].start()
            acc = contrib(acc, 0, h, lax.rem(my + (N_DEV - h), N_DEV))
            acc = contrib(acc, 1, h, lax.rem(my + h, N_DEV))

        for d in (0, 1):
            for j in range(E_HALF):
                rdmas[(d, N_DEV - 2, j)].wait_recv()
        acc = contrib(acc, 0, N_DEV - 1, lax.rem(my + 1, N_DEV))
        acc = contrib(acc, 1, N_DEV - 1, lax.rem(my + (N_DEV - 1), N_DEV))

        for (d, h, j), r in rdmas.items():
            r.wait_send()

        out_ref[:, :] = acc

    return pl.pallas_call(
        body,
        out_shape=jax.ShapeDtypeStruct((n_tok, d_ff), jnp.float32),
        in_specs=[pl.BlockSpec(memory_space=pltpu.VMEM)] * 5,
        out_specs=pl.BlockSpec(memory_space=pltpu.VMEM),
        scratch_shapes=[
            pltpu.VMEM((2, N_DEV, E_HALF, d_model, d_ff), jnp.bfloat16),
            pltpu.SemaphoreType.DMA((2, N_DEV - 1, E_HALF)),
            pltpu.SemaphoreType.DMA((2, N_DEV - 1, E_HALF)),
        ],
        compiler_params=pltpu.CompilerParams(collective_id=0),
    )(x, router_W, route_idx, expert_W, shared_W)
